# Initial kernel scaffold; baseline (speedup 1.0000x reference)
import jax
import jax.numpy as jnp
from jax.experimental import pallas as pl


def kernel(feature, edge_index, W_self, W_neigh, b, gamma, beta):
    raise NotImplementedError("write your pallas kernel here")


# speedup vs baseline: 4.4425x; 4.4425x over previous
import jax
import jax.numpy as jnp
from jax import lax
from jax.experimental import pallas as pl
from jax.experimental.pallas import tpu as pltpu
from jax.experimental.pallas import tpu_sc as plsc

N_ = 10000
E_ = 160000
D_ = 256
HALF_ = 128
NC_ = 2
NS_ = 16
CH_ = 128
EPT_ = 10112
NCHUNK_ = EPT_ // CH_
EPAD_ = EPT_ * NS_
ACC_ROWS_ = N_ + 8
DEGN_ = 10016
RPT_ = 624
WB_ = 16


def _sc_agg_body(feat_ref, src_ref, dst_ref, hn_out,
                 acc_sp, grid_sp, sidx, didx, rows, deg_v, dsum, dtmp,
                 buf, zbuf, sem):
    cid = lax.axis_index("c")
    tid = lax.axis_index("s")
    zero16 = jnp.zeros((16,), jnp.float32)
    one16 = jnp.ones((16,), jnp.float32)

    for r in range(8):
        for j in range(HALF_ // 16):
            zbuf[r, pl.ds(j * 16, 16)] = zero16

    def _zdeg(j, c):
        deg_v[pl.ds(j * 16, 16)] = zero16
        return c
    lax.fori_loop(0, DEGN_ // 16, _zdeg, 0)

    rbase = tid * RPT_

    def _z(k, c):
        pltpu.sync_copy(zbuf, acc_sp.at[pl.ds(rbase + k * 8, 8)])
        return c
    lax.fori_loop(0, RPT_ // 8, _z, 0)

    @pl.when(tid == NS_ - 1)
    def _():
        for k in range(3):
            pltpu.sync_copy(zbuf, acc_sp.at[pl.ds(NS_ * RPT_ + k * 8, 8)])

    plsc.subcore_barrier()

    ebase = tid * EPT_

    def _chunk(k, c):
        base = ebase + k * CH_
        pltpu.sync_copy(src_ref.at[pl.ds(cid * EPAD_ + base, CH_)], sidx)
        pltpu.sync_copy(dst_ref.at[pl.ds(base, CH_)], didx)
        for j in range(CH_ // 16):
            dv = didx[pl.ds(j * 16, 16)]
            plsc.addupdate_scatter(deg_v, [dv], one16)
        pltpu.async_copy(feat_ref.at[sidx], rows, sem).wait()
        pltpu.sync_copy(rows, acc_sp.at[didx], add=True)
        return c
    lax.fori_loop(0, NCHUNK_, _chunk, 0)

    pltpu.sync_copy(deg_v, grid_sp.at[pl.ds(pl.multiple_of(tid * DEGN_, 8),
                                            DEGN_)])
    plsc.subcore_barrier()

    nmine = 640

    def _zs(j, c):
        dsum[pl.ds(j * 16, 16)] = zero16
        return c
    lax.fori_loop(0, nmine // 16, _zs, 0)

    def _comb(g, c):
        pltpu.sync_copy(grid_sp.at[pl.ds(pl.multiple_of(g * DEGN_ + rbase, 8),
                                         nmine)], dtmp)

        def _acc(j, c2):
            dsum[pl.ds(j * 16, 16)] += dtmp[pl.ds(j * 16, 16)]
            return c2
        lax.fori_loop(0, nmine // 16, _acc, 0)
        return c
    lax.fori_loop(0, NS_, _comb, 0)

    def _rcp(j, c):
        v = dsum[pl.ds(j * 16, 16)]
        dsum[pl.ds(j * 16, 16)] = 1.0 / jnp.maximum(v, 1.0)
        return c
    lax.fori_loop(0, nmine // 16, _rcp, 0)

    def _wb(k, c):
        pltpu.sync_copy(acc_sp.at[pl.ds(rbase + k * WB_, WB_)], buf)
        sv = dsum[pl.ds(k * WB_, WB_)]
        for rr in range(WB_):
            s = sv[rr]
            for j in range(HALF_ // 16):
                buf[rr, pl.ds(j * 16, 16)] = buf[rr, pl.ds(j * 16, 16)] * s
        pltpu.sync_copy(buf, hn_out.at[pl.ds(cid * N_ + rbase + k * WB_, WB_)])
        return c
    lax.fori_loop(0, RPT_ // WB_, _wb, 0)

    @pl.when(tid == NS_ - 1)
    def _():
        _wb(RPT_ // WB_, 0)


_sc_agg = pl.kernel(
    _sc_agg_body,
    out_type=jax.ShapeDtypeStruct((NC_ * N_, HALF_), jnp.float32),
    mesh=plsc.VectorSubcoreMesh(core_axis_name="c", subcore_axis_name="s",
                                num_cores=NC_, num_subcores=NS_),
    compiler_params=pltpu.CompilerParams(needs_layout_passes=False),
    scratch_types=(
        pltpu.VMEM_SHARED((ACC_ROWS_, HALF_), jnp.float32),
        pltpu.VMEM_SHARED((NS_ * DEGN_,), jnp.float32),
        pltpu.VMEM((CH_,), jnp.int32),
        pltpu.VMEM((CH_,), jnp.int32),
        pltpu.VMEM((CH_, HALF_), jnp.float32),
        pltpu.VMEM((DEGN_,), jnp.float32),
        pltpu.VMEM((640,), jnp.float32),
        pltpu.VMEM((640,), jnp.float32),
        pltpu.VMEM((WB_, HALF_), jnp.float32),
        pltpu.VMEM((8, HALF_), jnp.float32),
        pltpu.SemaphoreType.DMA,
    ),
)

BLK_ = 1000
GRID_ = N_ // BLK_
_DN_ = (((1,), (1,)), ((), ()))


def _mm_body(x_ref, lo_ref, hi_ref, ws_ref, wn_ref, prm_ref, h_ref, st_ref):
    i = pl.program_id(0)
    x = x_ref[...]
    wn = wn_ref[...]
    h = lax.dot_general(x, ws_ref[...], _DN_,
                        precision=lax.Precision.HIGHEST,
                        preferred_element_type=jnp.float32)
    h = h + lax.dot_general(lo_ref[...], wn[:, :HALF_], _DN_,
                            precision=lax.Precision.HIGHEST,
                            preferred_element_type=jnp.float32)
    h = h + lax.dot_general(hi_ref[...], wn[:, HALF_:], _DN_,
                            precision=lax.Precision.HIGHEST,
                            preferred_element_type=jnp.float32)
    h = h + prm_ref[...][0:1, :]
    h_ref[...] = h
    s = jnp.sum(h, axis=0)
    sq = jnp.sum(h * h, axis=0)
    row = lax.broadcasted_iota(jnp.int32, (8, D_), 0)
    upd = (jnp.where(row == 0, s[None, :], 0.0)
           + jnp.where(row == 1, sq[None, :], 0.0))

    @pl.when(i == 0)
    def _():
        st_ref[...] = jnp.zeros((8, D_), jnp.float32)

    st_ref[...] += upd


def _bn_body(h_ref, x_ref, st_ref, prm_ref, o_ref):
    st = st_ref[...]
    mu = st[0:1, :] * (1.0 / N_)
    ex2 = st[1:2, :] * (1.0 / N_)
    var = ex2 - mu * mu
    inv = lax.rsqrt(var + 1e-5)
    g = prm_ref[...][1:2, :]
    be = prm_ref[...][2:3, :]
    y = (h_ref[...] - mu) * (inv * g) + be
    o_ref[...] = x_ref[...] + jnp.maximum(y, 0.0)


def kernel(feature, edge_index, W_self, W_neigh, b, gamma, beta):
    src = edge_index[0].astype(jnp.int32)
    dst = edge_index[1].astype(jnp.int32)
    npad = EPAD_ - E_
    pad_ar = jnp.arange(npad, dtype=jnp.int32)
    src_p = jnp.concatenate([src, pad_ar % N_])
    dst_p = jnp.concatenate([dst, N_ + (pad_ar % 8)])
    src2 = jnp.concatenate([2 * src_p, 2 * src_p + 1])
    feat2 = feature.reshape(2 * N_, HALF_)
    params = (jnp.zeros((8, D_), jnp.float32)
              .at[0].set(b).at[1].set(gamma).at[2].set(beta))

    hn = _sc_agg(feat2, src2, dst_p)

    h, stats = pl.pallas_call(
        _mm_body,
        grid=(GRID_,),
        in_specs=[
            pl.BlockSpec((BLK_, D_), lambda i: (i, 0)),
            pl.BlockSpec((BLK_, HALF_), lambda i: (i, 0)),
            pl.BlockSpec((BLK_, HALF_), lambda i: (GRID_ + i, 0)),
            pl.BlockSpec((D_, D_), lambda i: (0, 0)),
            pl.BlockSpec((D_, D_), lambda i: (0, 0)),
            pl.BlockSpec((8, D_), lambda i: (0, 0)),
        ],
        out_specs=[
            pl.BlockSpec((BLK_, D_), lambda i: (i, 0)),
            pl.BlockSpec((8, D_), lambda i: (0, 0)),
        ],
        out_shape=[jax.ShapeDtypeStruct((N_, D_), jnp.float32),
                   jax.ShapeDtypeStruct((8, D_), jnp.float32)],
    )(feature, hn, hn, W_self, W_neigh, params)

    out = pl.pallas_call(
        _bn_body,
        grid=(GRID_,),
        in_specs=[
            pl.BlockSpec((BLK_, D_), lambda i: (i, 0)),
            pl.BlockSpec((BLK_, D_), lambda i: (i, 0)),
            pl.BlockSpec((8, D_), lambda i: (0, 0)),
            pl.BlockSpec((8, D_), lambda i: (0, 0)),
        ],
        out_specs=pl.BlockSpec((BLK_, D_), lambda i: (i, 0)),
        out_shape=jax.ShapeDtypeStruct((N_, D_), jnp.float32),
    )(h, feature, stats, params)
    return out


# speedup vs baseline: 4.9468x; 1.1135x over previous
import jax
import jax.numpy as jnp
from jax import lax
from jax.experimental import pallas as pl
from jax.experimental.pallas import tpu as pltpu
from jax.experimental.pallas import tpu_sc as plsc

N_ = 10000
E_ = 160000
D_ = 256
HALF_ = 128
NC_ = 2
NS_ = 16
CH_ = 64
EPT_ = 10112
NCHUNK_ = EPT_ // CH_
EPAD_ = EPT_ * NS_
ACC_ROWS_ = N_ + 8
DEGN_ = 10016
RPT_ = 624
WB_ = 16


def _sc_agg_body(feat_ref, src_ref, dst_ref, hn_out,
                 acc_sp, grid_sp, sidx_a, didx_a, sidx_b, didx_b,
                 rows_a, rows_b, deg_v, dsum, dtmp,
                 buf, zbuf, gsem_a, gsem_b, ssem_a, ssem_b):
    cid = lax.axis_index("c")
    tid = lax.axis_index("s")
    zero16 = jnp.zeros((16,), jnp.float32)
    one16 = jnp.ones((16,), jnp.float32)

    for r in range(8):
        for j in range(HALF_ // 16):
            zbuf[r, pl.ds(j * 16, 16)] = zero16

    def _zdeg(j, c):
        deg_v[pl.ds(j * 16, 16)] = zero16
        return c
    lax.fori_loop(0, DEGN_ // 16, _zdeg, 0)

    rbase = tid * RPT_

    def _z(k, c):
        pltpu.sync_copy(zbuf, acc_sp.at[pl.ds(rbase + k * 8, 8)])
        return c
    lax.fori_loop(0, RPT_ // 8, _z, 0)

    @pl.when(tid == NS_ - 1)
    def _():
        for k in range(3):
            pltpu.sync_copy(zbuf, acc_sp.at[pl.ds(NS_ * RPT_ + k * 8, 8)])

    plsc.subcore_barrier()

    ebase = tid * EPT_

    def _load_idx(c, sidx, didx):
        base = ebase + c * CH_
        pltpu.sync_copy(src_ref.at[pl.ds(cid * EPAD_ + base, CH_)], sidx)
        pltpu.sync_copy(dst_ref.at[pl.ds(base, CH_)], didx)
        for j in range(CH_ // 16):
            dv = didx[pl.ds(j * 16, 16)]
            plsc.addupdate_scatter(deg_v, [dv], one16)

    _load_idx(0, sidx_a, didx_a)
    pltpu.async_copy(feat_ref.at[sidx_a], rows_a, gsem_a)

    def _super(s, c):
        c1 = 2 * s + 1
        c2 = 2 * s + 2

        @pl.when(s > 0)
        def _():
            pltpu.make_async_copy(rows_b, acc_sp.at[didx_b], ssem_b).wait()
        _load_idx(c1, sidx_b, didx_b)
        pltpu.make_async_copy(feat_ref.at[sidx_a], rows_a, gsem_a).wait()
        pltpu.async_copy(rows_a, acc_sp.at[didx_a], ssem_a, add=True)
        pltpu.async_copy(feat_ref.at[sidx_b], rows_b, gsem_b)
        pltpu.make_async_copy(rows_a, acc_sp.at[didx_a], ssem_a).wait()

        @pl.when(c2 < NCHUNK_)
        def _():
            _load_idx(c2, sidx_a, didx_a)
        pltpu.make_async_copy(feat_ref.at[sidx_b], rows_b, gsem_b).wait()
        pltpu.async_copy(rows_b, acc_sp.at[didx_b], ssem_b, add=True)

        @pl.when(c2 < NCHUNK_)
        def _():
            pltpu.async_copy(feat_ref.at[sidx_a], rows_a, gsem_a)
        return c
    lax.fori_loop(0, NCHUNK_ // 2, _super, 0)
    pltpu.make_async_copy(rows_b, acc_sp.at[didx_b], ssem_b).wait()

    pltpu.sync_copy(deg_v, grid_sp.at[pl.ds(pl.multiple_of(tid * DEGN_, 8),
                                            DEGN_)])
    plsc.subcore_barrier()

    nmine = 640

    def _zs(j, c):
        dsum[pl.ds(j * 16, 16)] = zero16
        return c
    lax.fori_loop(0, nmine // 16, _zs, 0)

    def _comb(g, c):
        pltpu.sync_copy(grid_sp.at[pl.ds(pl.multiple_of(g * DEGN_ + rbase, 8),
                                         nmine)], dtmp)

        def _acc(j, c2):
            dsum[pl.ds(j * 16, 16)] += dtmp[pl.ds(j * 16, 16)]
            return c2
        lax.fori_loop(0, nmine // 16, _acc, 0)
        return c
    lax.fori_loop(0, NS_, _comb, 0)

    def _rcp(j, c):
        v = dsum[pl.ds(j * 16, 16)]
        dsum[pl.ds(j * 16, 16)] = 1.0 / jnp.maximum(v, 1.0)
        return c
    lax.fori_loop(0, nmine // 16, _rcp, 0)

    def _wb(k, c):
        pltpu.sync_copy(acc_sp.at[pl.ds(rbase + k * WB_, WB_)], buf)
        sv = dsum[pl.ds(k * WB_, WB_)]
        for rr in range(WB_):
            s = sv[rr]
            for j in range(HALF_ // 16):
                buf[rr, pl.ds(j * 16, 16)] = buf[rr, pl.ds(j * 16, 16)] * s
        pltpu.sync_copy(buf, hn_out.at[pl.ds(cid * N_ + rbase + k * WB_, WB_)])
        return c
    lax.fori_loop(0, RPT_ // WB_, _wb, 0)

    @pl.when(tid == NS_ - 1)
    def _():
        _wb(RPT_ // WB_, 0)


_sc_agg = pl.kernel(
    _sc_agg_body,
    out_type=jax.ShapeDtypeStruct((NC_ * N_, HALF_), jnp.float32),
    mesh=plsc.VectorSubcoreMesh(core_axis_name="c", subcore_axis_name="s",
                                num_cores=NC_, num_subcores=NS_),
    compiler_params=pltpu.CompilerParams(needs_layout_passes=False),
    scratch_types=(
        pltpu.VMEM_SHARED((ACC_ROWS_, HALF_), jnp.float32),
        pltpu.VMEM_SHARED((NS_ * DEGN_,), jnp.float32),
        pltpu.VMEM((CH_,), jnp.int32),
        pltpu.VMEM((CH_,), jnp.int32),
        pltpu.VMEM((CH_,), jnp.int32),
        pltpu.VMEM((CH_,), jnp.int32),
        pltpu.VMEM((CH_, HALF_), jnp.float32),
        pltpu.VMEM((CH_, HALF_), jnp.float32),
        pltpu.VMEM((DEGN_,), jnp.float32),
        pltpu.VMEM((640,), jnp.float32),
        pltpu.VMEM((640,), jnp.float32),
        pltpu.VMEM((WB_, HALF_), jnp.float32),
        pltpu.VMEM((8, HALF_), jnp.float32),
        pltpu.SemaphoreType.DMA,
        pltpu.SemaphoreType.DMA,
        pltpu.SemaphoreType.DMA,
        pltpu.SemaphoreType.DMA,
    ),
)

BLK_ = 1000
GRID_ = N_ // BLK_
_DN_ = (((1,), (1,)), ((), ()))


def _mm_body(x_ref, lo_ref, hi_ref, ws_ref, wn_ref, prm_ref, h_ref, st_ref):
    i = pl.program_id(0)
    x = x_ref[...]
    wn = wn_ref[...]
    h = lax.dot_general(x, ws_ref[...], _DN_,
                        precision=lax.Precision.HIGHEST,
                        preferred_element_type=jnp.float32)
    h = h + lax.dot_general(lo_ref[...], wn[:, :HALF_], _DN_,
                            precision=lax.Precision.HIGHEST,
                            preferred_element_type=jnp.float32)
    h = h + lax.dot_general(hi_ref[...], wn[:, HALF_:], _DN_,
                            precision=lax.Precision.HIGHEST,
                            preferred_element_type=jnp.float32)
    h = h + prm_ref[...][0:1, :]
    h_ref[...] = h
    s = jnp.sum(h, axis=0)
    sq = jnp.sum(h * h, axis=0)
    row = lax.broadcasted_iota(jnp.int32, (8, D_), 0)
    upd = (jnp.where(row == 0, s[None, :], 0.0)
           + jnp.where(row == 1, sq[None, :], 0.0))

    @pl.when(i == 0)
    def _():
        st_ref[...] = jnp.zeros((8, D_), jnp.float32)

    st_ref[...] += upd


def _bn_body(h_ref, x_ref, st_ref, prm_ref, o_ref):
    st = st_ref[...]
    mu = st[0:1, :] * (1.0 / N_)
    ex2 = st[1:2, :] * (1.0 / N_)
    var = ex2 - mu * mu
    inv = lax.rsqrt(var + 1e-5)
    g = prm_ref[...][1:2, :]
    be = prm_ref[...][2:3, :]
    y = (h_ref[...] - mu) * (inv * g) + be
    o_ref[...] = x_ref[...] + jnp.maximum(y, 0.0)


def kernel(feature, edge_index, W_self, W_neigh, b, gamma, beta):
    src = edge_index[0].astype(jnp.int32)
    dst = edge_index[1].astype(jnp.int32)
    npad = EPAD_ - E_
    pad_ar = jnp.arange(npad, dtype=jnp.int32)
    src_p = jnp.concatenate([src, pad_ar % N_])
    dst_p = jnp.concatenate([dst, N_ + (pad_ar % 8)])
    src2 = jnp.concatenate([2 * src_p, 2 * src_p + 1])
    feat2 = feature.reshape(2 * N_, HALF_)
    params = (jnp.zeros((8, D_), jnp.float32)
              .at[0].set(b).at[1].set(gamma).at[2].set(beta))

    hn = _sc_agg(feat2, src2, dst_p)

    h, stats = pl.pallas_call(
        _mm_body,
        grid=(GRID_,),
        in_specs=[
            pl.BlockSpec((BLK_, D_), lambda i: (i, 0)),
            pl.BlockSpec((BLK_, HALF_), lambda i: (i, 0)),
            pl.BlockSpec((BLK_, HALF_), lambda i: (GRID_ + i, 0)),
            pl.BlockSpec((D_, D_), lambda i: (0, 0)),
            pl.BlockSpec((D_, D_), lambda i: (0, 0)),
            pl.BlockSpec((8, D_), lambda i: (0, 0)),
        ],
        out_specs=[
            pl.BlockSpec((BLK_, D_), lambda i: (i, 0)),
            pl.BlockSpec((8, D_), lambda i: (0, 0)),
        ],
        out_shape=[jax.ShapeDtypeStruct((N_, D_), jnp.float32),
                   jax.ShapeDtypeStruct((8, D_), jnp.float32)],
    )(feature, hn, hn, W_self, W_neigh, params)

    out = pl.pallas_call(
        _bn_body,
        grid=(GRID_,),
        in_specs=[
            pl.BlockSpec((BLK_, D_), lambda i: (i, 0)),
            pl.BlockSpec((BLK_, D_), lambda i: (i, 0)),
            pl.BlockSpec((8, D_), lambda i: (0, 0)),
            pl.BlockSpec((8, D_), lambda i: (0, 0)),
        ],
        out_specs=pl.BlockSpec((BLK_, D_), lambda i: (i, 0)),
        out_shape=jax.ShapeDtypeStruct((N_, D_), jnp.float32),
    )(h, feature, stats, params)
    return out


# speedup vs baseline: 5.5665x; 1.1253x over previous
import jax
import jax.numpy as jnp
from jax import lax
from jax.experimental import pallas as pl
from jax.experimental.pallas import tpu as pltpu
from jax.experimental.pallas import tpu_sc as plsc

N_ = 10000
E_ = 160000
D_ = 256
HALF_ = 128
NC_ = 2
NS_ = 16
CH_ = 64
EPT_ = 10112
NCHUNK_ = EPT_ // CH_
EPAD_ = EPT_ * NS_
ACC_ROWS_ = N_ + 8
DEGN_ = 10016
RPT_ = 624
WB_ = 16


def _sc_agg_body(feat_ref, src_ref, dst_ref, hn_out,
                 acc_sp, grid_sp, sidx_a, didx_a, sidx_b, didx_b,
                 rows_a, rows_b, deg_v, dsum, dtmp,
                 buf, zbuf, gsem_a, gsem_b, ssem_a, ssem_b,
                 isem_sa, isem_da, isem_sb, isem_db):
    cid = lax.axis_index("c")
    tid = lax.axis_index("s")
    zero16 = jnp.zeros((16,), jnp.float32)
    one16 = jnp.ones((16,), jnp.float32)

    for r in range(8):
        for j in range(HALF_ // 16):
            zbuf[r, pl.ds(j * 16, 16)] = zero16

    def _zdeg(j, c):
        deg_v[pl.ds(j * 16, 16)] = zero16
        return c
    lax.fori_loop(0, DEGN_ // 16, _zdeg, 0)

    rbase = tid * RPT_

    def _z(k, c):
        pltpu.sync_copy(zbuf, acc_sp.at[pl.ds(rbase + k * 8, 8)])
        return c
    lax.fori_loop(0, RPT_ // 8, _z, 0)

    @pl.when(tid == NS_ - 1)
    def _():
        for k in range(3):
            pltpu.sync_copy(zbuf, acc_sp.at[pl.ds(NS_ * RPT_ + k * 8, 8)])

    plsc.subcore_barrier()

    ebase = tid * EPT_

    def _start_idx(c, sidx, didx, sem_s, sem_d):
        base = ebase + c * CH_
        pltpu.async_copy(src_ref.at[pl.ds(cid * EPAD_ + base, CH_)],
                         sidx, sem_s)
        pltpu.async_copy(dst_ref.at[pl.ds(base, CH_)], didx, sem_d)

    def _wait_idx(c, sidx, didx, sem_s, sem_d):
        base = ebase + c * CH_
        pltpu.make_async_copy(src_ref.at[pl.ds(cid * EPAD_ + base, CH_)],
                              sidx, sem_s).wait()
        pltpu.make_async_copy(dst_ref.at[pl.ds(base, CH_)],
                              didx, sem_d).wait()
        for j in range(CH_ // 16):
            dv = didx[pl.ds(j * 16, 16)]
            plsc.addupdate_scatter(deg_v, [dv], one16)

    _start_idx(0, sidx_a, didx_a, isem_sa, isem_da)
    _wait_idx(0, sidx_a, didx_a, isem_sa, isem_da)
    pltpu.async_copy(feat_ref.at[sidx_a], rows_a, gsem_a)
    _start_idx(1, sidx_b, didx_b, isem_sb, isem_db)

    def _super(s, c):
        c1 = 2 * s + 1
        c2 = 2 * s + 2

        @pl.when(s > 0)
        def _():
            pltpu.make_async_copy(rows_b, acc_sp.at[didx_b], ssem_b).wait()
            _start_idx(c1, sidx_b, didx_b, isem_sb, isem_db)
        pltpu.make_async_copy(feat_ref.at[sidx_a], rows_a, gsem_a).wait()
        pltpu.async_copy(rows_a, acc_sp.at[didx_a], ssem_a, add=True)
        _wait_idx(c1, sidx_b, didx_b, isem_sb, isem_db)
        pltpu.async_copy(feat_ref.at[sidx_b], rows_b, gsem_b)
        pltpu.make_async_copy(rows_a, acc_sp.at[didx_a], ssem_a).wait()

        @pl.when(c2 < NCHUNK_)
        def _():
            _start_idx(c2, sidx_a, didx_a, isem_sa, isem_da)
        pltpu.make_async_copy(feat_ref.at[sidx_b], rows_b, gsem_b).wait()
        pltpu.async_copy(rows_b, acc_sp.at[didx_b], ssem_b, add=True)

        @pl.when(c2 < NCHUNK_)
        def _():
            _wait_idx(c2, sidx_a, didx_a, isem_sa, isem_da)
            pltpu.async_copy(feat_ref.at[sidx_a], rows_a, gsem_a)
        return c
    lax.fori_loop(0, NCHUNK_ // 2, _super, 0)
    pltpu.make_async_copy(rows_b, acc_sp.at[didx_b], ssem_b).wait()

    pltpu.sync_copy(deg_v, grid_sp.at[pl.ds(pl.multiple_of(tid * DEGN_, 8),
                                            DEGN_)])
    plsc.subcore_barrier()

    nmine = 640

    def _zs(j, c):
        dsum[pl.ds(j * 16, 16)] = zero16
        return c
    lax.fori_loop(0, nmine // 16, _zs, 0)

    def _comb(g, c):
        pltpu.sync_copy(grid_sp.at[pl.ds(pl.multiple_of(g * DEGN_ + rbase, 8),
                                         nmine)], dtmp)

        def _acc(j, c2):
            dsum[pl.ds(j * 16, 16)] += dtmp[pl.ds(j * 16, 16)]
            return c2
        lax.fori_loop(0, nmine // 16, _acc, 0)
        return c
    lax.fori_loop(0, NS_, _comb, 0)

    def _rcp(j, c):
        v = dsum[pl.ds(j * 16, 16)]
        dsum[pl.ds(j * 16, 16)] = 1.0 / jnp.maximum(v, 1.0)
        return c
    lax.fori_loop(0, nmine // 16, _rcp, 0)

    def _wb(k, c):
        pltpu.sync_copy(acc_sp.at[pl.ds(rbase + k * WB_, WB_)], buf)
        sv = dsum[pl.ds(k * WB_, WB_)]
        for rr in range(WB_):
            s = sv[rr]
            for j in range(HALF_ // 16):
                buf[rr, pl.ds(j * 16, 16)] = buf[rr, pl.ds(j * 16, 16)] * s
        pltpu.sync_copy(buf, hn_out.at[pl.ds(cid * N_ + rbase + k * WB_, WB_)])
        return c
    lax.fori_loop(0, RPT_ // WB_, _wb, 0)

    @pl.when(tid == NS_ - 1)
    def _():
        _wb(RPT_ // WB_, 0)


_sc_agg = pl.kernel(
    _sc_agg_body,
    out_type=jax.ShapeDtypeStruct((NC_ * N_, HALF_), jnp.float32),
    mesh=plsc.VectorSubcoreMesh(core_axis_name="c", subcore_axis_name="s",
                                num_cores=NC_, num_subcores=NS_),
    compiler_params=pltpu.CompilerParams(needs_layout_passes=False),
    scratch_types=(
        pltpu.VMEM_SHARED((ACC_ROWS_, HALF_), jnp.float32),
        pltpu.VMEM_SHARED((NS_ * DEGN_,), jnp.float32),
        pltpu.VMEM((CH_,), jnp.int32),
        pltpu.VMEM((CH_,), jnp.int32),
        pltpu.VMEM((CH_,), jnp.int32),
        pltpu.VMEM((CH_,), jnp.int32),
        pltpu.VMEM((CH_, HALF_), jnp.float32),
        pltpu.VMEM((CH_, HALF_), jnp.float32),
        pltpu.VMEM((DEGN_,), jnp.float32),
        pltpu.VMEM((640,), jnp.float32),
        pltpu.VMEM((640,), jnp.float32),
        pltpu.VMEM((WB_, HALF_), jnp.float32),
        pltpu.VMEM((8, HALF_), jnp.float32),
        pltpu.SemaphoreType.DMA,
        pltpu.SemaphoreType.DMA,
        pltpu.SemaphoreType.DMA,
        pltpu.SemaphoreType.DMA,
        pltpu.SemaphoreType.DMA,
        pltpu.SemaphoreType.DMA,
        pltpu.SemaphoreType.DMA,
        pltpu.SemaphoreType.DMA,
    ),
)

BLK_ = 1000
GRID_ = N_ // BLK_
_DN_ = (((1,), (1,)), ((), ()))


def _mm_body(x_ref, lo_ref, hi_ref, ws_ref, wn_ref, prm_ref, h_ref, st_ref):
    i = pl.program_id(0)
    x = x_ref[...]
    wn = wn_ref[...]
    h = lax.dot_general(x, ws_ref[...], _DN_,
                        precision=lax.Precision.HIGHEST,
                        preferred_element_type=jnp.float32)
    h = h + lax.dot_general(lo_ref[...], wn[:, :HALF_], _DN_,
                            precision=lax.Precision.HIGHEST,
                            preferred_element_type=jnp.float32)
    h = h + lax.dot_general(hi_ref[...], wn[:, HALF_:], _DN_,
                            precision=lax.Precision.HIGHEST,
                            preferred_element_type=jnp.float32)
    h = h + prm_ref[...][0:1, :]
    h_ref[...] = h
    s = jnp.sum(h, axis=0)
    sq = jnp.sum(h * h, axis=0)
    row = lax.broadcasted_iota(jnp.int32, (8, D_), 0)
    upd = (jnp.where(row == 0, s[None, :], 0.0)
           + jnp.where(row == 1, sq[None, :], 0.0))

    @pl.when(i == 0)
    def _():
        st_ref[...] = jnp.zeros((8, D_), jnp.float32)

    st_ref[...] += upd


def _bn_body(h_ref, x_ref, st_ref, prm_ref, o_ref):
    st = st_ref[...]
    mu = st[0:1, :] * (1.0 / N_)
    ex2 = st[1:2, :] * (1.0 / N_)
    var = ex2 - mu * mu
    inv = lax.rsqrt(var + 1e-5)
    g = prm_ref[...][1:2, :]
    be = prm_ref[...][2:3, :]
    y = (h_ref[...] - mu) * (inv * g) + be
    o_ref[...] = x_ref[...] + jnp.maximum(y, 0.0)


def kernel(feature, edge_index, W_self, W_neigh, b, gamma, beta):
    src = edge_index[0].astype(jnp.int32)
    dst = edge_index[1].astype(jnp.int32)
    npad = EPAD_ - E_
    pad_ar = jnp.arange(npad, dtype=jnp.int32)
    src_p = jnp.concatenate([src, pad_ar % N_])
    dst_p = jnp.concatenate([dst, N_ + (pad_ar % 8)])
    src2 = jnp.concatenate([2 * src_p, 2 * src_p + 1])
    feat2 = feature.reshape(2 * N_, HALF_)
    params = (jnp.zeros((8, D_), jnp.float32)
              .at[0].set(b).at[1].set(gamma).at[2].set(beta))

    hn = _sc_agg(feat2, src2, dst_p)

    h, stats = pl.pallas_call(
        _mm_body,
        grid=(GRID_,),
        in_specs=[
            pl.BlockSpec((BLK_, D_), lambda i: (i, 0)),
            pl.BlockSpec((BLK_, HALF_), lambda i: (i, 0)),
            pl.BlockSpec((BLK_, HALF_), lambda i: (GRID_ + i, 0)),
            pl.BlockSpec((D_, D_), lambda i: (0, 0)),
            pl.BlockSpec((D_, D_), lambda i: (0, 0)),
            pl.BlockSpec((8, D_), lambda i: (0, 0)),
        ],
        out_specs=[
            pl.BlockSpec((BLK_, D_), lambda i: (i, 0)),
            pl.BlockSpec((8, D_), lambda i: (0, 0)),
        ],
        out_shape=[jax.ShapeDtypeStruct((N_, D_), jnp.float32),
                   jax.ShapeDtypeStruct((8, D_), jnp.float32)],
    )(feature, hn, hn, W_self, W_neigh, params)

    out = pl.pallas_call(
        _bn_body,
        grid=(GRID_,),
        in_specs=[
            pl.BlockSpec((BLK_, D_), lambda i: (i, 0)),
            pl.BlockSpec((BLK_, D_), lambda i: (i, 0)),
            pl.BlockSpec((8, D_), lambda i: (0, 0)),
            pl.BlockSpec((8, D_), lambda i: (0, 0)),
        ],
        out_specs=pl.BlockSpec((BLK_, D_), lambda i: (i, 0)),
        out_shape=jax.ShapeDtypeStruct((N_, D_), jnp.float32),
    )(h, feature, stats, params)
    return out


# speedup vs baseline: 6.6185x; 1.1890x over previous
import jax
import jax.numpy as jnp
from jax import lax
from jax.experimental import pallas as pl
from jax.experimental.pallas import tpu as pltpu
from jax.experimental.pallas import tpu_sc as plsc

N_ = 10000
E_ = 160000
D_ = 256
HALF_ = 128
NC_ = 2
NS_ = 16
CH_ = 128
EPT_ = 10240
NCHUNK_ = EPT_ // CH_
EPAD_ = EPT_ * NS_
ACC_ROWS_ = N_ + 8
DEGN_ = 10016
RPT_ = 624
WB_ = 16


def _sc_agg_body(feat_ref, src_ref, dst_ref, hn_out,
                 acc_sp, grid_sp, sidx_a, didx_a, sidx_b, didx_b,
                 rows_a, rows_b, deg_v, dsum, dtmp,
                 gsem_a, gsem_b, ssem_a, ssem_b,
                 isem_sa, isem_da, isem_sb, isem_db):
    cid = lax.axis_index("c")
    tid = lax.axis_index("s")
    zero16 = jnp.zeros((16,), jnp.float32)
    one16 = jnp.ones((16,), jnp.float32)

    for r in range(8):
        for j in range(HALF_ // 16):
            rows_a[r, pl.ds(j * 16, 16)] = zero16

    def _zdeg(j, c):
        deg_v[pl.ds(j * 16, 16)] = zero16
        return c
    lax.fori_loop(0, DEGN_ // 16, _zdeg, 0)

    rbase = tid * RPT_
    zsrc = rows_a.at[pl.ds(0, 8)]

    def _z(k, c):
        pltpu.sync_copy(zsrc, acc_sp.at[pl.ds(rbase + k * 8, 8)])
        return c
    lax.fori_loop(0, RPT_ // 8, _z, 0)

    @pl.when(tid == NS_ - 1)
    def _():
        for k in range(3):
            pltpu.sync_copy(zsrc, acc_sp.at[pl.ds(NS_ * RPT_ + k * 8, 8)])

    plsc.subcore_barrier()

    ebase = tid * EPT_

    def _start_idx(c, sidx, didx, sem_s, sem_d):
        base = ebase + c * CH_
        pltpu.async_copy(src_ref.at[pl.ds(cid * EPAD_ + base, CH_)],
                         sidx, sem_s)
        pltpu.async_copy(dst_ref.at[pl.ds(base, CH_)], didx, sem_d)

    def _wait_idx(c, sidx, didx, sem_s, sem_d):
        base = ebase + c * CH_
        pltpu.make_async_copy(src_ref.at[pl.ds(cid * EPAD_ + base, CH_)],
                              sidx, sem_s).wait()
        pltpu.make_async_copy(dst_ref.at[pl.ds(base, CH_)],
                              didx, sem_d).wait()
        for j in range(CH_ // 16):
            dv = didx[pl.ds(j * 16, 16)]
            plsc.addupdate_scatter(deg_v, [dv], one16)

    _start_idx(0, sidx_a, didx_a, isem_sa, isem_da)
    _wait_idx(0, sidx_a, didx_a, isem_sa, isem_da)
    pltpu.async_copy(feat_ref.at[sidx_a], rows_a, gsem_a)
    _start_idx(1, sidx_b, didx_b, isem_sb, isem_db)

    def _super(s, c):
        c1 = 2 * s + 1
        c2 = 2 * s + 2

        @pl.when(s > 0)
        def _():
            pltpu.make_async_copy(rows_b, acc_sp.at[didx_b], ssem_b).wait()
            _start_idx(c1, sidx_b, didx_b, isem_sb, isem_db)
        pltpu.make_async_copy(feat_ref.at[sidx_a], rows_a, gsem_a).wait()
        pltpu.async_copy(rows_a, acc_sp.at[didx_a], ssem_a, add=True)
        _wait_idx(c1, sidx_b, didx_b, isem_sb, isem_db)
        pltpu.async_copy(feat_ref.at[sidx_b], rows_b, gsem_b)
        pltpu.make_async_copy(rows_a, acc_sp.at[didx_a], ssem_a).wait()

        @pl.when(c2 < NCHUNK_)
        def _():
            _start_idx(c2, sidx_a, didx_a, isem_sa, isem_da)
        pltpu.make_async_copy(feat_ref.at[sidx_b], rows_b, gsem_b).wait()
        pltpu.async_copy(rows_b, acc_sp.at[didx_b], ssem_b, add=True)

        @pl.when(c2 < NCHUNK_)
        def _():
            _wait_idx(c2, sidx_a, didx_a, isem_sa, isem_da)
            pltpu.async_copy(feat_ref.at[sidx_a], rows_a, gsem_a)
        return c
    lax.fori_loop(0, NCHUNK_ // 2, _super, 0)
    pltpu.make_async_copy(rows_b, acc_sp.at[didx_b], ssem_b).wait()

    nmine = 640

    def _zs(j, c):
        dsum[pl.ds(j * 16, 16)] = zero16
        return c
    lax.fori_loop(0, nmine // 16, _zs, 0)

    for ph in range(2):
        @pl.when((tid >= ph * 8) & (tid < ph * 8 + 8))
        def _():
            off = pl.multiple_of((tid - ph * 8) * DEGN_, 8)
            pltpu.sync_copy(deg_v, grid_sp.at[pl.ds(off, DEGN_)])
        plsc.subcore_barrier()

        def _comb(g, c):
            pltpu.sync_copy(
                grid_sp.at[pl.ds(pl.multiple_of(g * DEGN_ + rbase, 8),
                                 nmine)], dtmp)

            def _acc(j, c2):
                dsum[pl.ds(j * 16, 16)] += dtmp[pl.ds(j * 16, 16)]
                return c2
            lax.fori_loop(0, nmine // 16, _acc, 0)
            return c
        lax.fori_loop(0, 8, _comb, 0)
        plsc.subcore_barrier()

    def _rcp(j, c):
        v = dsum[pl.ds(j * 16, 16)]
        dsum[pl.ds(j * 16, 16)] = 1.0 / jnp.maximum(v, 1.0)
        return c
    lax.fori_loop(0, nmine // 16, _rcp, 0)

    buf = rows_a.at[pl.ds(0, WB_)]

    def _wb(k, c):
        pltpu.sync_copy(acc_sp.at[pl.ds(rbase + k * WB_, WB_)], buf)
        sv = dsum[pl.ds(k * WB_, WB_)]
        for rr in range(WB_):
            s = sv[rr]
            for j in range(HALF_ // 16):
                buf[rr, pl.ds(j * 16, 16)] = buf[rr, pl.ds(j * 16, 16)] * s
        pltpu.sync_copy(buf, hn_out.at[pl.ds(cid * N_ + rbase + k * WB_, WB_)])
        return c
    lax.fori_loop(0, RPT_ // WB_, _wb, 0)

    @pl.when(tid == NS_ - 1)
    def _():
        _wb(RPT_ // WB_, 0)


_sc_agg = pl.kernel(
    _sc_agg_body,
    out_type=jax.ShapeDtypeStruct((NC_ * N_, HALF_), jnp.float32),
    mesh=plsc.VectorSubcoreMesh(core_axis_name="c", subcore_axis_name="s",
                                num_cores=NC_, num_subcores=NS_),
    compiler_params=pltpu.CompilerParams(needs_layout_passes=False),
    scratch_types=(
        pltpu.VMEM_SHARED((ACC_ROWS_, HALF_), jnp.float32),
        pltpu.VMEM_SHARED((8 * DEGN_,), jnp.float32),
        pltpu.VMEM((CH_,), jnp.int32),
        pltpu.VMEM((CH_,), jnp.int32),
        pltpu.VMEM((CH_,), jnp.int32),
        pltpu.VMEM((CH_,), jnp.int32),
        pltpu.VMEM((CH_, HALF_), jnp.float32),
        pltpu.VMEM((CH_, HALF_), jnp.float32),
        pltpu.VMEM((DEGN_,), jnp.float32),
        pltpu.VMEM((640,), jnp.float32),
        pltpu.VMEM((640,), jnp.float32),
        pltpu.SemaphoreType.DMA,
        pltpu.SemaphoreType.DMA,
        pltpu.SemaphoreType.DMA,
        pltpu.SemaphoreType.DMA,
        pltpu.SemaphoreType.DMA,
        pltpu.SemaphoreType.DMA,
        pltpu.SemaphoreType.DMA,
        pltpu.SemaphoreType.DMA,
    ),
)

BLK_ = 1000
GRID_ = N_ // BLK_
_DN_ = (((1,), (1,)), ((), ()))


def _mm_body(x_ref, lo_ref, hi_ref, ws_ref, wn_ref, prm_ref, h_ref, st_ref):
    i = pl.program_id(0)
    x = x_ref[...]
    wn = wn_ref[...]
    h = lax.dot_general(x, ws_ref[...], _DN_,
                        precision=lax.Precision.HIGHEST,
                        preferred_element_type=jnp.float32)
    h = h + lax.dot_general(lo_ref[...], wn[:, :HALF_], _DN_,
                            precision=lax.Precision.HIGHEST,
                            preferred_element_type=jnp.float32)
    h = h + lax.dot_general(hi_ref[...], wn[:, HALF_:], _DN_,
                            precision=lax.Precision.HIGHEST,
                            preferred_element_type=jnp.float32)
    h = h + prm_ref[...][0:1, :]
    h_ref[...] = h
    s = jnp.sum(h, axis=0)
    sq = jnp.sum(h * h, axis=0)
    row = lax.broadcasted_iota(jnp.int32, (8, D_), 0)
    upd = (jnp.where(row == 0, s[None, :], 0.0)
           + jnp.where(row == 1, sq[None, :], 0.0))

    @pl.when(i == 0)
    def _():
        st_ref[...] = jnp.zeros((8, D_), jnp.float32)

    st_ref[...] += upd


def _bn_body(h_ref, x_ref, st_ref, prm_ref, o_ref):
    st = st_ref[...]
    mu = st[0:1, :] * (1.0 / N_)
    ex2 = st[1:2, :] * (1.0 / N_)
    var = ex2 - mu * mu
    inv = lax.rsqrt(var + 1e-5)
    g = prm_ref[...][1:2, :]
    be = prm_ref[...][2:3, :]
    y = (h_ref[...] - mu) * (inv * g) + be
    o_ref[...] = x_ref[...] + jnp.maximum(y, 0.0)


def kernel(feature, edge_index, W_self, W_neigh, b, gamma, beta):
    src = edge_index[0].astype(jnp.int32)
    dst = edge_index[1].astype(jnp.int32)
    npad = EPAD_ - E_
    pad_ar = jnp.arange(npad, dtype=jnp.int32)
    src_p = jnp.concatenate([src, pad_ar % N_])
    dst_p = jnp.concatenate([dst, N_ + (pad_ar % 8)])
    src2 = jnp.concatenate([2 * src_p, 2 * src_p + 1])
    feat2 = feature.reshape(2 * N_, HALF_)
    params = (jnp.zeros((8, D_), jnp.float32)
              .at[0].set(b).at[1].set(gamma).at[2].set(beta))

    hn = _sc_agg(feat2, src2, dst_p)

    h, stats = pl.pallas_call(
        _mm_body,
        grid=(GRID_,),
        in_specs=[
            pl.BlockSpec((BLK_, D_), lambda i: (i, 0)),
            pl.BlockSpec((BLK_, HALF_), lambda i: (i, 0)),
            pl.BlockSpec((BLK_, HALF_), lambda i: (GRID_ + i, 0)),
            pl.BlockSpec((D_, D_), lambda i: (0, 0)),
            pl.BlockSpec((D_, D_), lambda i: (0, 0)),
            pl.BlockSpec((8, D_), lambda i: (0, 0)),
        ],
        out_specs=[
            pl.BlockSpec((BLK_, D_), lambda i: (i, 0)),
            pl.BlockSpec((8, D_), lambda i: (0, 0)),
        ],
        out_shape=[jax.ShapeDtypeStruct((N_, D_), jnp.float32),
                   jax.ShapeDtypeStruct((8, D_), jnp.float32)],
    )(feature, hn, hn, W_self, W_neigh, params)

    out = pl.pallas_call(
        _bn_body,
        grid=(GRID_,),
        in_specs=[
            pl.BlockSpec((BLK_, D_), lambda i: (i, 0)),
            pl.BlockSpec((BLK_, D_), lambda i: (i, 0)),
            pl.BlockSpec((8, D_), lambda i: (0, 0)),
            pl.BlockSpec((8, D_), lambda i: (0, 0)),
        ],
        out_specs=pl.BlockSpec((BLK_, D_), lambda i: (i, 0)),
        out_shape=jax.ShapeDtypeStruct((N_, D_), jnp.float32),
    )(h, feature, stats, params)
    return out


# speedup vs baseline: 6.8626x; 1.0369x over previous
import jax
import jax.numpy as jnp
from jax import lax
from jax.experimental import pallas as pl
from jax.experimental.pallas import tpu as pltpu
from jax.experimental.pallas import tpu_sc as plsc

N_ = 10000
E_ = 160000
D_ = 256
HALF_ = 128
NC_ = 2
NS_ = 16
CH_ = 128
EPT_ = 10240
NCHUNK_ = EPT_ // CH_
EPAD_ = EPT_ * NS_
ACC_ROWS_ = N_ + 8
DEGN_ = 10016
RPT_ = 624
WB_ = 16


def _sc_agg_body(feat_ref, src_ref, dst_ref, hn_out,
                 acc_sp, grid_sp, sidx_a, didx_a, sidx_b, didx_b,
                 rows_a, rows_b, deg_v, dsum, dtmp,
                 gsem_a, gsem_b, ssem_a, ssem_b,
                 isem_sa, isem_da, isem_sb, isem_db):
    cid = lax.axis_index("c")
    tid = lax.axis_index("s")
    zero16 = jnp.zeros((16,), jnp.float32)
    one16 = jnp.ones((16,), jnp.float32)

    for r in range(8):
        for j in range(HALF_ // 16):
            rows_a[r, pl.ds(j * 16, 16)] = zero16

    def _zdeg(j, c):
        deg_v[pl.ds(j * 16, 16)] = zero16
        return c
    lax.fori_loop(0, DEGN_ // 16, _zdeg, 0)

    rbase = tid * RPT_
    zsrc = rows_a.at[pl.ds(0, 8)]

    def _z(k, c):
        pltpu.sync_copy(zsrc, acc_sp.at[pl.ds(rbase + k * 8, 8)])
        return c
    lax.fori_loop(0, RPT_ // 8, _z, 0)

    @pl.when(tid == NS_ - 1)
    def _():
        for k in range(3):
            pltpu.sync_copy(zsrc, acc_sp.at[pl.ds(NS_ * RPT_ + k * 8, 8)])

    plsc.subcore_barrier()

    ebase = tid * EPT_

    def _start_idx(c, sidx, didx, sem_s, sem_d):
        base = ebase + c * CH_
        pltpu.async_copy(src_ref.at[pl.ds(cid * EPAD_ + base, CH_)],
                         sidx, sem_s)
        pltpu.async_copy(dst_ref.at[pl.ds(base, CH_)], didx, sem_d)

    def _wait_idx(c, sidx, didx, sem_s, sem_d):
        base = ebase + c * CH_
        pltpu.make_async_copy(src_ref.at[pl.ds(cid * EPAD_ + base, CH_)],
                              sidx, sem_s).wait()
        pltpu.make_async_copy(dst_ref.at[pl.ds(base, CH_)],
                              didx, sem_d).wait()
        for j in range(CH_ // 16):
            dv = didx[pl.ds(j * 16, 16)]
            plsc.addupdate_scatter(deg_v, [dv], one16)

    _start_idx(0, sidx_a, didx_a, isem_sa, isem_da)
    _wait_idx(0, sidx_a, didx_a, isem_sa, isem_da)
    pltpu.async_copy(feat_ref.at[sidx_a], rows_a, gsem_a)
    _start_idx(1, sidx_b, didx_b, isem_sb, isem_db)

    def _super(s, c):
        c1 = 2 * s + 1
        c2 = 2 * s + 2

        @pl.when(s > 0)
        def _():
            pltpu.make_async_copy(rows_b, acc_sp.at[didx_b], ssem_b).wait()
            _start_idx(c1, sidx_b, didx_b, isem_sb, isem_db)
        pltpu.make_async_copy(feat_ref.at[sidx_a], rows_a, gsem_a).wait()
        pltpu.async_copy(rows_a, acc_sp.at[didx_a], ssem_a, add=True)
        _wait_idx(c1, sidx_b, didx_b, isem_sb, isem_db)
        pltpu.async_copy(feat_ref.at[sidx_b], rows_b, gsem_b)
        pltpu.make_async_copy(rows_a, acc_sp.at[didx_a], ssem_a).wait()

        @pl.when(c2 < NCHUNK_)
        def _():
            _start_idx(c2, sidx_a, didx_a, isem_sa, isem_da)
        pltpu.make_async_copy(feat_ref.at[sidx_b], rows_b, gsem_b).wait()
        pltpu.async_copy(rows_b, acc_sp.at[didx_b], ssem_b, add=True)

        @pl.when(c2 < NCHUNK_)
        def _():
            _wait_idx(c2, sidx_a, didx_a, isem_sa, isem_da)
            pltpu.async_copy(feat_ref.at[sidx_a], rows_a, gsem_a)
        return c
    lax.fori_loop(0, NCHUNK_ // 2, _super, 0)
    pltpu.make_async_copy(rows_b, acc_sp.at[didx_b], ssem_b).wait()

    nmine = 640

    def _zs(j, c):
        dsum[pl.ds(j * 16, 16)] = zero16
        return c
    lax.fori_loop(0, nmine // 16, _zs, 0)

    for ph in range(2):
        @pl.when((tid >= ph * 8) & (tid < ph * 8 + 8))
        def _():
            off = pl.multiple_of((tid - ph * 8) * DEGN_, 8)
            pltpu.sync_copy(deg_v, grid_sp.at[pl.ds(off, DEGN_)])
        plsc.subcore_barrier()

        def _comb(g, c):
            pltpu.sync_copy(
                grid_sp.at[pl.ds(pl.multiple_of(g * DEGN_ + rbase, 8),
                                 nmine)], dtmp)

            def _acc(j, c2):
                dsum[pl.ds(j * 16, 16)] += dtmp[pl.ds(j * 16, 16)]
                return c2
            lax.fori_loop(0, nmine // 16, _acc, 0)
            return c
        lax.fori_loop(0, 8, _comb, 0)
        plsc.subcore_barrier()

    def _rcp(j, c):
        v = dsum[pl.ds(j * 16, 16)]
        dsum[pl.ds(j * 16, 16)] = 1.0 / jnp.maximum(v, 1.0)
        return c
    lax.fori_loop(0, nmine // 16, _rcp, 0)

    buf = rows_a.at[pl.ds(0, WB_)]

    def _wb(k, c):
        pltpu.sync_copy(acc_sp.at[pl.ds(rbase + k * WB_, WB_)], buf)
        sv = dsum[pl.ds(k * WB_, WB_)]
        for rr in range(WB_):
            s = sv[rr]
            for j in range(HALF_ // 16):
                buf[rr, pl.ds(j * 16, 16)] = buf[rr, pl.ds(j * 16, 16)] * s
        pltpu.sync_copy(buf, hn_out.at[pl.ds(cid * N_ + rbase + k * WB_, WB_)])
        return c
    lax.fori_loop(0, RPT_ // WB_, _wb, 0)

    @pl.when(tid == NS_ - 1)
    def _():
        _wb(RPT_ // WB_, 0)


_sc_agg = pl.kernel(
    _sc_agg_body,
    out_type=jax.ShapeDtypeStruct((NC_ * N_, HALF_), jnp.float32),
    mesh=plsc.VectorSubcoreMesh(core_axis_name="c", subcore_axis_name="s",
                                num_cores=NC_, num_subcores=NS_),
    compiler_params=pltpu.CompilerParams(needs_layout_passes=False),
    scratch_types=(
        pltpu.VMEM_SHARED((ACC_ROWS_, HALF_), jnp.float32),
        pltpu.VMEM_SHARED((8 * DEGN_,), jnp.float32),
        pltpu.VMEM((CH_,), jnp.int32),
        pltpu.VMEM((CH_,), jnp.int32),
        pltpu.VMEM((CH_,), jnp.int32),
        pltpu.VMEM((CH_,), jnp.int32),
        pltpu.VMEM((CH_, HALF_), jnp.float32),
        pltpu.VMEM((CH_, HALF_), jnp.float32),
        pltpu.VMEM((DEGN_,), jnp.float32),
        pltpu.VMEM((640,), jnp.float32),
        pltpu.VMEM((640,), jnp.float32),
        pltpu.SemaphoreType.DMA,
        pltpu.SemaphoreType.DMA,
        pltpu.SemaphoreType.DMA,
        pltpu.SemaphoreType.DMA,
        pltpu.SemaphoreType.DMA,
        pltpu.SemaphoreType.DMA,
        pltpu.SemaphoreType.DMA,
        pltpu.SemaphoreType.DMA,
    ),
)

BLK_ = 1000
GRID_ = N_ // BLK_
_DN_ = (((1,), (1,)), ((), ()))


def _mm_self_body(x_ref, ws_ref, prm_ref, hs_ref):
    hs = lax.dot_general(x_ref[...], ws_ref[...], _DN_,
                         precision=lax.Precision.HIGHEST,
                         preferred_element_type=jnp.float32)
    hs_ref[...] = hs + prm_ref[...][0:1, :]


def _mm_body(hs_ref, lo_ref, hi_ref, wn_ref, h_ref, st_ref):
    i = pl.program_id(0)
    wn = wn_ref[...]
    h = hs_ref[...]
    h = h + lax.dot_general(lo_ref[...], wn[:, :HALF_], _DN_,
                            precision=lax.Precision.HIGHEST,
                            preferred_element_type=jnp.float32)
    h = h + lax.dot_general(hi_ref[...], wn[:, HALF_:], _DN_,
                            precision=lax.Precision.HIGHEST,
                            preferred_element_type=jnp.float32)
    h_ref[...] = h
    s = jnp.sum(h, axis=0)
    sq = jnp.sum(h * h, axis=0)
    row = lax.broadcasted_iota(jnp.int32, (8, D_), 0)
    upd = (jnp.where(row == 0, s[None, :], 0.0)
           + jnp.where(row == 1, sq[None, :], 0.0))

    @pl.when(i == 0)
    def _():
        st_ref[...] = jnp.zeros((8, D_), jnp.float32)

    st_ref[...] += upd


def _bn_body(h_ref, x_ref, st_ref, prm_ref, o_ref):
    st = st_ref[...]
    mu = st[0:1, :] * (1.0 / N_)
    ex2 = st[1:2, :] * (1.0 / N_)
    var = ex2 - mu * mu
    inv = lax.rsqrt(var + 1e-5)
    g = prm_ref[...][1:2, :]
    be = prm_ref[...][2:3, :]
    y = (h_ref[...] - mu) * (inv * g) + be
    o_ref[...] = x_ref[...] + jnp.maximum(y, 0.0)


def kernel(feature, edge_index, W_self, W_neigh, b, gamma, beta):
    src = edge_index[0].astype(jnp.int32)
    dst = edge_index[1].astype(jnp.int32)
    npad = EPAD_ - E_
    pad_ar = jnp.arange(npad, dtype=jnp.int32)
    src_p = jnp.concatenate([src, pad_ar % N_])
    dst_p = jnp.concatenate([dst, N_ + (pad_ar % 8)])
    src2 = jnp.concatenate([2 * src_p, 2 * src_p + 1])
    feat2 = feature.reshape(2 * N_, HALF_)
    params = (jnp.zeros((8, D_), jnp.float32)
              .at[0].set(b).at[1].set(gamma).at[2].set(beta))

    hn = _sc_agg(feat2, src2, dst_p)

    hs = pl.pallas_call(
        _mm_self_body,
        grid=(GRID_,),
        in_specs=[
            pl.BlockSpec((BLK_, D_), lambda i: (i, 0)),
            pl.BlockSpec((D_, D_), lambda i: (0, 0)),
            pl.BlockSpec((8, D_), lambda i: (0, 0)),
        ],
        out_specs=pl.BlockSpec((BLK_, D_), lambda i: (i, 0)),
        out_shape=jax.ShapeDtypeStruct((N_, D_), jnp.float32),
    )(feature, W_self, params)

    h, stats = pl.pallas_call(
        _mm_body,
        grid=(GRID_,),
        in_specs=[
            pl.BlockSpec((BLK_, D_), lambda i: (i, 0)),
            pl.BlockSpec((BLK_, HALF_), lambda i: (i, 0)),
            pl.BlockSpec((BLK_, HALF_), lambda i: (GRID_ + i, 0)),
            pl.BlockSpec((D_, D_), lambda i: (0, 0)),
        ],
        out_specs=[
            pl.BlockSpec((BLK_, D_), lambda i: (i, 0)),
            pl.BlockSpec((8, D_), lambda i: (0, 0)),
        ],
        out_shape=[jax.ShapeDtypeStruct((N_, D_), jnp.float32),
                   jax.ShapeDtypeStruct((8, D_), jnp.float32)],
    )(hs, hn, hn, W_neigh)

    out = pl.pallas_call(
        _bn_body,
        grid=(GRID_,),
        in_specs=[
            pl.BlockSpec((BLK_, D_), lambda i: (i, 0)),
            pl.BlockSpec((BLK_, D_), lambda i: (i, 0)),
            pl.BlockSpec((8, D_), lambda i: (0, 0)),
            pl.BlockSpec((8, D_), lambda i: (0, 0)),
        ],
        out_specs=pl.BlockSpec((BLK_, D_), lambda i: (i, 0)),
        out_shape=jax.ShapeDtypeStruct((N_, D_), jnp.float32),
    )(h, feature, stats, params)
    return out


# speedup vs baseline: 6.9145x; 1.0076x over previous
import jax
import jax.numpy as jnp
from jax import lax
from jax.experimental import pallas as pl
from jax.experimental.pallas import tpu as pltpu
from jax.experimental.pallas import tpu_sc as plsc

N_ = 10000
E_ = 160000
D_ = 256
HALF_ = 128
NC_ = 2
NS_ = 16
CH_ = 128
EPT_ = 10240
NCHUNK_ = EPT_ // CH_
EPAD_ = EPT_ * NS_
ACC_ROWS_ = N_ + 8
DEGN_ = 10016
RPT_ = 624
WB_ = 16


def _sc_agg_body(feat_ref, src_ref, dst_ref, hn_out,
                 acc_sp, grid_sp, sidx_a, didx_a, sidx_b, didx_b,
                 rows_a, rows_b, deg_v, dsum, dtmp,
                 gsem_a, gsem_b, ssem_a, ssem_b,
                 isem_sa, isem_da, isem_sb, isem_db):
    cid = lax.axis_index("c")
    tid = lax.axis_index("s")
    zero16 = jnp.zeros((16,), jnp.float32)
    one16 = jnp.ones((16,), jnp.float32)

    for r in range(8):
        for j in range(HALF_ // 16):
            rows_a[r, pl.ds(j * 16, 16)] = zero16

    def _zdeg(j, c):
        deg_v[pl.ds(j * 16, 16)] = zero16
        return c
    lax.fori_loop(0, DEGN_ // 16, _zdeg, 0)

    rbase = tid * RPT_
    zsrc = rows_a.at[pl.ds(0, 8)]

    def _z(k, c):
        pltpu.sync_copy(zsrc, acc_sp.at[pl.ds(rbase + k * 8, 8)])
        return c
    lax.fori_loop(0, RPT_ // 8, _z, 0)

    @pl.when(tid == NS_ - 1)
    def _():
        for k in range(3):
            pltpu.sync_copy(zsrc, acc_sp.at[pl.ds(NS_ * RPT_ + k * 8, 8)])

    plsc.subcore_barrier()

    ebase = tid * EPT_

    def _start_idx(c, sidx, didx, sem_s, sem_d):
        base = ebase + c * CH_
        pltpu.async_copy(src_ref.at[pl.ds(cid * EPAD_ + base, CH_)],
                         sidx, sem_s)
        pltpu.async_copy(dst_ref.at[pl.ds(base, CH_)], didx, sem_d)

    def _wait_idx(c, sidx, didx, sem_s, sem_d):
        base = ebase + c * CH_
        pltpu.make_async_copy(src_ref.at[pl.ds(cid * EPAD_ + base, CH_)],
                              sidx, sem_s).wait()
        pltpu.make_async_copy(dst_ref.at[pl.ds(base, CH_)],
                              didx, sem_d).wait()
        for j in range(CH_ // 16):
            dv = didx[pl.ds(j * 16, 16)]
            plsc.addupdate_scatter(deg_v, [dv], one16)

    _start_idx(0, sidx_a, didx_a, isem_sa, isem_da)
    _wait_idx(0, sidx_a, didx_a, isem_sa, isem_da)
    pltpu.async_copy(feat_ref.at[sidx_a], rows_a, gsem_a)
    _start_idx(1, sidx_b, didx_b, isem_sb, isem_db)

    def _super(s, c):
        c1 = 2 * s + 1
        c2 = 2 * s + 2

        @pl.when(s > 0)
        def _():
            pltpu.make_async_copy(rows_b, acc_sp.at[didx_b], ssem_b).wait()
            _start_idx(c1, sidx_b, didx_b, isem_sb, isem_db)
        pltpu.make_async_copy(feat_ref.at[sidx_a], rows_a, gsem_a).wait()
        pltpu.async_copy(rows_a, acc_sp.at[didx_a], ssem_a, add=True)
        _wait_idx(c1, sidx_b, didx_b, isem_sb, isem_db)
        pltpu.async_copy(feat_ref.at[sidx_b], rows_b, gsem_b)
        pltpu.make_async_copy(rows_a, acc_sp.at[didx_a], ssem_a).wait()

        @pl.when(c2 < NCHUNK_)
        def _():
            _start_idx(c2, sidx_a, didx_a, isem_sa, isem_da)
        pltpu.make_async_copy(feat_ref.at[sidx_b], rows_b, gsem_b).wait()
        pltpu.async_copy(rows_b, acc_sp.at[didx_b], ssem_b, add=True)

        @pl.when(c2 < NCHUNK_)
        def _():
            _wait_idx(c2, sidx_a, didx_a, isem_sa, isem_da)
            pltpu.async_copy(feat_ref.at[sidx_a], rows_a, gsem_a)
        return c
    lax.fori_loop(0, NCHUNK_ // 2, _super, 0)
    pltpu.make_async_copy(rows_b, acc_sp.at[didx_b], ssem_b).wait()

    nmine = 640

    def _zs(j, c):
        dsum[pl.ds(j * 16, 16)] = zero16
        return c
    lax.fori_loop(0, nmine // 16, _zs, 0)

    for ph in range(2):
        @pl.when((tid >= ph * 8) & (tid < ph * 8 + 8))
        def _():
            off = pl.multiple_of((tid - ph * 8) * DEGN_, 8)
            pltpu.sync_copy(deg_v, grid_sp.at[pl.ds(off, DEGN_)])
        plsc.subcore_barrier()

        def _comb(g, c):
            pltpu.sync_copy(
                grid_sp.at[pl.ds(pl.multiple_of(g * DEGN_ + rbase, 8),
                                 nmine)], dtmp)

            def _acc(j, c2):
                dsum[pl.ds(j * 16, 16)] += dtmp[pl.ds(j * 16, 16)]
                return c2
            lax.fori_loop(0, nmine // 16, _acc, 0)
            return c
        lax.fori_loop(0, 8, _comb, 0)
        plsc.subcore_barrier()

    def _rcp(j, c):
        v = dsum[pl.ds(j * 16, 16)]
        dsum[pl.ds(j * 16, 16)] = 1.0 / jnp.maximum(v, 1.0)
        return c
    lax.fori_loop(0, nmine // 16, _rcp, 0)

    buf = rows_a.at[pl.ds(0, WB_)]

    def _wb(k, c):
        pltpu.sync_copy(acc_sp.at[pl.ds(rbase + k * WB_, WB_)], buf)
        sv = dsum[pl.ds(k * WB_, WB_)]
        for rr in range(WB_):
            s = sv[rr]
            for j in range(HALF_ // 16):
                buf[rr, pl.ds(j * 16, 16)] = buf[rr, pl.ds(j * 16, 16)] * s
        pltpu.sync_copy(buf, hn_out.at[pl.ds(cid * N_ + rbase + k * WB_, WB_)])
        return c
    lax.fori_loop(0, RPT_ // WB_, _wb, 0)

    @pl.when(tid == NS_ - 1)
    def _():
        _wb(RPT_ // WB_, 0)


_sc_agg = pl.kernel(
    _sc_agg_body,
    out_type=jax.ShapeDtypeStruct((NC_ * N_, HALF_), jnp.float32),
    mesh=plsc.VectorSubcoreMesh(core_axis_name="c", subcore_axis_name="s",
                                num_cores=NC_, num_subcores=NS_),
    compiler_params=pltpu.CompilerParams(needs_layout_passes=False),
    scratch_types=(
        pltpu.VMEM_SHARED((ACC_ROWS_, HALF_), jnp.float32),
        pltpu.VMEM_SHARED((8 * DEGN_,), jnp.float32),
        pltpu.VMEM((CH_,), jnp.int32),
        pltpu.VMEM((CH_,), jnp.int32),
        pltpu.VMEM((CH_,), jnp.int32),
        pltpu.VMEM((CH_,), jnp.int32),
        pltpu.VMEM((CH_, HALF_), jnp.float32),
        pltpu.VMEM((CH_, HALF_), jnp.float32),
        pltpu.VMEM((DEGN_,), jnp.float32),
        pltpu.VMEM((640,), jnp.float32),
        pltpu.VMEM((640,), jnp.float32),
        pltpu.SemaphoreType.DMA,
        pltpu.SemaphoreType.DMA,
        pltpu.SemaphoreType.DMA,
        pltpu.SemaphoreType.DMA,
        pltpu.SemaphoreType.DMA,
        pltpu.SemaphoreType.DMA,
        pltpu.SemaphoreType.DMA,
        pltpu.SemaphoreType.DMA,
    ),
)

BLK_ = 2000
GRID_ = N_ // BLK_
_DN_ = (((1,), (1,)), ((), ()))


def _mm_self_body(x_ref, ws_ref, prm_ref, hs_ref):
    hs = lax.dot_general(x_ref[...], ws_ref[...], _DN_,
                         precision=lax.Precision.HIGHEST,
                         preferred_element_type=jnp.float32)
    hs_ref[...] = hs + prm_ref[...][0:1, :]


def _mm_body(hs_ref, lo_ref, hi_ref, wn_ref, h_ref, st_ref):
    i = pl.program_id(0)
    wn = wn_ref[...]
    h = hs_ref[...]
    h = h + lax.dot_general(lo_ref[...], wn[:, :HALF_], _DN_,
                            precision=lax.Precision.HIGHEST,
                            preferred_element_type=jnp.float32)
    h = h + lax.dot_general(hi_ref[...], wn[:, HALF_:], _DN_,
                            precision=lax.Precision.HIGHEST,
                            preferred_element_type=jnp.float32)
    h_ref[...] = h
    s = jnp.sum(h, axis=0)
    sq = jnp.sum(h * h, axis=0)
    row = lax.broadcasted_iota(jnp.int32, (8, D_), 0)
    upd = (jnp.where(row == 0, s[None, :], 0.0)
           + jnp.where(row == 1, sq[None, :], 0.0))

    @pl.when(i == 0)
    def _():
        st_ref[...] = jnp.zeros((8, D_), jnp.float32)

    st_ref[...] += upd


def _bn_body(h_ref, x_ref, st_ref, prm_ref, o_ref):
    st = st_ref[...]
    mu = st[0:1, :] * (1.0 / N_)
    ex2 = st[1:2, :] * (1.0 / N_)
    var = ex2 - mu * mu
    inv = lax.rsqrt(var + 1e-5)
    g = prm_ref[...][1:2, :]
    be = prm_ref[...][2:3, :]
    y = (h_ref[...] - mu) * (inv * g) + be
    o_ref[...] = x_ref[...] + jnp.maximum(y, 0.0)


def kernel(feature, edge_index, W_self, W_neigh, b, gamma, beta):
    src = edge_index[0].astype(jnp.int32)
    dst = edge_index[1].astype(jnp.int32)
    npad = EPAD_ - E_
    pad_ar = jnp.arange(npad, dtype=jnp.int32)
    src_p = jnp.concatenate([src, pad_ar % N_])
    dst_p = jnp.concatenate([dst, N_ + (pad_ar % 8)])
    src2 = jnp.concatenate([2 * src_p, 2 * src_p + 1])
    feat2 = feature.reshape(2 * N_, HALF_)
    params = (jnp.zeros((8, D_), jnp.float32)
              .at[0].set(b).at[1].set(gamma).at[2].set(beta))

    hn = _sc_agg(feat2, src2, dst_p)

    hs = pl.pallas_call(
        _mm_self_body,
        grid=(GRID_,),
        in_specs=[
            pl.BlockSpec((BLK_, D_), lambda i: (i, 0)),
            pl.BlockSpec((D_, D_), lambda i: (0, 0)),
            pl.BlockSpec((8, D_), lambda i: (0, 0)),
        ],
        out_specs=pl.BlockSpec((BLK_, D_), lambda i: (i, 0)),
        out_shape=jax.ShapeDtypeStruct((N_, D_), jnp.float32),
    )(feature, W_self, params)

    h, stats = pl.pallas_call(
        _mm_body,
        grid=(GRID_,),
        in_specs=[
            pl.BlockSpec((BLK_, D_), lambda i: (i, 0)),
            pl.BlockSpec((BLK_, HALF_), lambda i: (i, 0)),
            pl.BlockSpec((BLK_, HALF_), lambda i: (GRID_ + i, 0)),
            pl.BlockSpec((D_, D_), lambda i: (0, 0)),
        ],
        out_specs=[
            pl.BlockSpec((BLK_, D_), lambda i: (i, 0)),
            pl.BlockSpec((8, D_), lambda i: (0, 0)),
        ],
        out_shape=[jax.ShapeDtypeStruct((N_, D_), jnp.float32),
                   jax.ShapeDtypeStruct((8, D_), jnp.float32)],
    )(hs, hn, hn, W_neigh)

    out = pl.pallas_call(
        _bn_body,
        grid=(GRID_,),
        in_specs=[
            pl.BlockSpec((BLK_, D_), lambda i: (i, 0)),
            pl.BlockSpec((BLK_, D_), lambda i: (i, 0)),
            pl.BlockSpec((8, D_), lambda i: (0, 0)),
            pl.BlockSpec((8, D_), lambda i: (0, 0)),
        ],
        out_specs=pl.BlockSpec((BLK_, D_), lambda i: (i, 0)),
        out_shape=jax.ShapeDtypeStruct((N_, D_), jnp.float32),
    )(h, feature, stats, params)
    return out
